# baseline (speedup 1.0000x reference)
import functools

import jax
import jax.numpy as jnp
from jax import lax
from jax.experimental import pallas as pl
from jax.experimental.pallas import tpu as pltpu
from jax.experimental.pallas import tpu_sc as plsc

E = 160000
MNN = 6
IN = 128
H = 128
HEADS = 8
DPH = 16

NEG = -1e18
SLOPE = 0.01


_B1 = 1600


def _lrelu(v):
    return jnp.where(v >= 0, v, SLOPE * v)


def _tc1_body(x_ref, wzx_ref, wr_ref, whx_ref, whh_ref, wk_ref, wv_ref, wq_ref,
              cz_ref, cr_ref, whb_ref, wkb_ref, wvb_ref, wqb_ref, sqb_ref,
              aq_ref, ak_ref, seg_ref, v0_ref, sks_ref, sq_ref):
    x = x_ref[...]
    f32 = jnp.float32
    dot = functools.partial(jnp.dot, preferred_element_type=f32)
    wvb = wvb_ref[...]
    z0 = jax.nn.sigmoid(dot(x, wzx_ref[...]) + cz_ref[...])
    r0 = jax.nn.sigmoid(dot(x, wr_ref[...]) + cr_ref[...])
    pre0 = jnp.tanh(dot(x, whx_ref[...]) + dot(r0 * wvb, whh_ref[...]) + whb_ref[...])
    h0 = (1.0 - z0) * wvb + z0 * pre0
    rid = lax.broadcasted_iota(jnp.int32, (x.shape[0], 1), 0) + pl.program_id(0) * _B1
    h0 = jnp.where(rid == 0, f32(0.0), h0)
    v0_ref[...] = dot(h0, wv_ref[...]) + wvb
    k0 = dot(h0, wk_ref[...]) + wkb_ref[...]
    sk = dot(_lrelu(k0) * ak_ref[...], seg_ref[...])
    s0 = jnp.sum(h0, axis=1, keepdims=True)
    pad7 = jnp.zeros((x.shape[0], 7), f32)
    sks_ref[...] = jnp.concatenate([sk, s0, pad7], axis=1)
    q = dot(x, wq_ref[...]) + wqb_ref[...]
    sq = dot(_lrelu(q) * aq_ref[...], seg_ref[...]) + sqb_ref[...]
    pad8 = jnp.zeros((x.shape[0], 8), f32)
    sq_ref[...] = jnp.concatenate([sq, pad8], axis=1)


def _tc1(x, wzxT, wrT, whxT, whhT, wkT, wvT, wqT, cz, cr, whb, wkb, wvb, wqb,
         sqb, aq, ak, seg):
    nblk = E // _B1
    row_blk = pl.BlockSpec((_B1, H), lambda i: (i, 0))
    full = lambda shape: pl.BlockSpec(shape, lambda i: tuple(0 for _ in shape))
    return pl.pallas_call(
        _tc1_body,
        grid=(nblk,),
        in_specs=[row_blk] + [full((H, H))] * 7 + [full((1, H))] * 6
                 + [full((1, HEADS))] + [full((1, H))] * 2 + [full((H, HEADS))],
        out_specs=[row_blk,
                   pl.BlockSpec((_B1, 16), lambda i: (i, 0)),
                   pl.BlockSpec((_B1, 16), lambda i: (i, 0))],
        out_shape=[jax.ShapeDtypeStruct((E, H), jnp.float32),
                   jax.ShapeDtypeStruct((E, 16), jnp.float32),
                   jax.ShapeDtypeStruct((E, 16), jnp.float32)],
    )(x, wzxT, wrT, whxT, whhT, wkT, wvT, wqT, cz, cr, whb, wkb, wvb, wqb,
      sqb, aq, ak, seg)


_NC, _NS = 2, 16
_NW = _NC * _NS
_R = 16
_EPC = _R * MNN
_NCHUNK = E // _R


_G = 8
_BR = _G * _R
_BE = _G * _EPC
_NBATCH = E // _BR


def _sc_body(bg_hbm, v0_hbm, sks_hbm, sq_hbm, out_hbm,
             idx_v, sq_v, outb_v, vg_v, sksg_v, sems):
    cid = lax.axis_index("c")
    sid = lax.axis_index("s")
    wid = sid * _NC + cid
    nb = (_NBATCH - wid + _NW - 1) // _NW
    i16 = lax.iota(jnp.int32, 16)
    rows = [i16 * MNN + j for j in range(MNN)]
    col_s0 = jnp.full((16,), HEADS, jnp.int32)

    def issue(g, p):
        ids = idx_v.at[pl.ds(g * _EPC, _EPC)]
        cv = pltpu.async_copy(v0_hbm.at[ids], vg_v.at[p], sems.at[2 * p])
        cs = pltpu.async_copy(sks_hbm.at[ids], sksg_v.at[p], sems.at[2 * p + 1])
        return cv, cs

    def drain(g, p):
        ids = idx_v.at[pl.ds(g * _EPC, _EPC)]
        pltpu.make_async_copy(v0_hbm.at[ids], vg_v.at[p], sems.at[2 * p]).wait()
        pltpu.make_async_copy(sks_hbm.at[ids], sksg_v.at[p], sems.at[2 * p + 1]).wait()

    _dnums = lax.GatherDimensionNumbers(offset_dims=(), collapsed_slice_dims=(0,),
                                        start_index_map=(0,))

    def splat(vec, lane):
        idx = jnp.full((16, 1), lane, jnp.int32)
        return lax.gather(vec, idx, _dnums, (1,),
                          mode=lax.GatherScatterMode.PROMISE_IN_BOUNDS)

    cols = [jnp.int32(h * DPH) + i16 for h in range(HEADS)]

    def compute(g, p):
        vg = vg_v.at[p]
        sksg = sksg_v.at[p]

        @plsc.parallel_loop(0, _R, unroll=2)
        def row_body(r):
            row = g * _R + r
            rowspl = jnp.broadcast_to(row, (16,)).astype(jnp.int32)
            sqv = plsc.load_gather(sq_v, [rowspl, i16])
            erows = [jnp.broadcast_to(r * MNN + j, (16,)).astype(jnp.int32)
                     for j in range(MNN)]
            kv = [plsc.load_gather(sksg, [erows[j], i16]) for j in range(MNN)]
            sc = [jnp.where(splat(kv[j], HEADS) == 0.0, NEG, sqv + kv[j])
                  for j in range(MNN)]
            m = sc[0]
            for j in range(1, MNN):
                m = jnp.maximum(m, sc[j])
            es = [jnp.exp(s - m) for s in sc]
            tot = es[0]
            for j in range(1, MNN):
                tot = tot + es[j]
            inv = 1.0 / tot
            wv = [e * inv for e in es]
            for h in range(HEADS):
                acc = None
                for j in range(MNN):
                    wsp = splat(wv[j], h)
                    vvec = plsc.load_gather(vg, [erows[j], cols[h]])
                    acc = wsp * vvec if acc is None else acc + wsp * vvec
                plsc.store_scatter(outb_v, [rowspl, cols[h]], acc)

    def batch(b, carry):
        bid = wid + _NW * b
        base_row = bid * _BR
        base_edge = bid * _BE
        pltpu.sync_copy(bg_hbm.at[pl.ds(base_edge, _BE)], idx_v)
        pltpu.sync_copy(sq_hbm.at[pl.ds(base_row, _BR), :], sq_v)
        issue(0, 0)

        def gpair(gp, carry2):
            g0 = gp * 2
            issue(g0 + 1, 1)
            drain(g0, 0)
            compute(g0, 0)

            @pl.when(gp < _G // 2 - 1)
            def _():
                issue(g0 + 2, 0)

            drain(g0 + 1, 1)
            compute(g0 + 1, 1)
            return carry2

        lax.fori_loop(0, _G // 2, gpair, 0)
        pltpu.sync_copy(outb_v, out_hbm.at[pl.ds(base_row, _BR), :])
        return carry

    lax.fori_loop(0, nb, batch, 0)


def _sc_attn(bg_flat, v0, sks, sq):
    mesh = plsc.VectorSubcoreMesh(core_axis_name="c", subcore_axis_name="s",
                                  num_cores=_NC, num_subcores=_NS)
    f = pl.kernel(
        _sc_body,
        out_type=jax.ShapeDtypeStruct((E, H), jnp.float32),
        mesh=mesh,
        compiler_params=pltpu.CompilerParams(needs_layout_passes=False,
                                             use_tc_tiling_on_sc=False),
        scratch_types=[
            pltpu.VMEM((_BE,), jnp.int32),
            pltpu.VMEM((_BR, 16), jnp.float32),
            pltpu.VMEM((_BR, H), jnp.float32),
            pltpu.VMEM((2, _EPC, H), jnp.float32),
            pltpu.VMEM((2, _EPC, 16), jnp.float32),
            pltpu.SemaphoreType.DMA((4,)),
        ],
    )
    return f(bg_flat, v0, sks, sq)


def _tc2_body(x_ref, sh_ref, wzx_ref, wzh_ref, wr_ref, ur_ref, whx_ref,
              whh_ref, wzb_ref, urb_ref, whb_ref, out_ref):
    x = x_ref[...]
    sh = sh_ref[...]
    dot = functools.partial(jnp.dot, preferred_element_type=jnp.float32)
    z = jax.nn.sigmoid(dot(x, wzx_ref[...]) + dot(sh, wzh_ref[...]) + wzb_ref[...])
    r = jax.nn.sigmoid(dot(x, wr_ref[...]) + dot(sh, ur_ref[...]) + urb_ref[...])
    pre = jnp.tanh(dot(x, whx_ref[...]) + dot(r * sh, whh_ref[...]) + whb_ref[...])
    h = (1.0 - z) * sh + z * pre
    rid = lax.broadcasted_iota(jnp.int32, (x.shape[0], 1), 0) + pl.program_id(0) * _B1
    out_ref[...] = jnp.where(rid == 0, jnp.float32(0.0), h)


def _tc2(x, sum_h, wzxT, wzhT, wrT, urT, whxT, whhT, wzb, urb, whb):
    nblk = E // _B1
    row_blk = pl.BlockSpec((_B1, H), lambda i: (i, 0))
    full = lambda shape: pl.BlockSpec(shape, lambda i: tuple(0 for _ in shape))
    return pl.pallas_call(
        _tc2_body,
        grid=(nblk,),
        in_specs=[row_blk, row_blk] + [full((H, H))] * 6 + [full((1, H))] * 3,
        out_specs=row_blk,
        out_shape=jax.ShapeDtypeStruct((E, H), jnp.float32),
    )(x, sum_h, wzxT, wzhT, wrT, urT, whxT, whhT, wzb, urb, whb)


def kernel(fmess, bgraph, W_z_w, W_z_b, W_r_w, U_r_w, U_r_b, W_h_w, W_h_b,
           attn_alpha, attn_bias, Wq_w, Wq_b, Wk_w, Wk_b, Wv_w, Wv_b):
    x = fmess
    f32 = jnp.float32

    Wz_x, Wz_h = W_z_w[:, :IN], W_z_w[:, IN:]
    Wh_x, Wh_h = W_h_w[:, :IN], W_h_w[:, IN:]
    alpha = attn_alpha.reshape(HEADS, 2 * DPH)
    aq = alpha[:, :DPH].reshape(1, H)
    ak = alpha[:, DPH:].reshape(1, H)
    bias = attn_bias.reshape(1, HEADS)

    col = jnp.arange(H)
    seg = (col[:, None] // DPH == jnp.arange(HEADS)[None, :]).astype(f32)

    cz = (Wv_b @ Wz_h.T + W_z_b).reshape(1, H)
    cr = (Wv_b @ U_r_w.T + U_r_b).reshape(1, H)

    v0, sks, sq = _tc1(
        x, Wz_x.T, W_r_w.T, Wh_x.T, Wh_h.T, Wk_w.T, Wv_w.T, Wq_w.T,
        cz, cr, W_h_b.reshape(1, H), Wk_b.reshape(1, H), Wv_b.reshape(1, H),
        Wq_b.reshape(1, H), bias, aq, ak, seg)

    sum_h = _sc_attn(bgraph.reshape(E * MNN), v0, sks, sq)

    return _tc2(x, sum_h, Wz_x.T, Wz_h.T, W_r_w.T, U_r_w.T, Wh_x.T, Wh_h.T,
                W_z_b.reshape(1, H), U_r_b.reshape(1, H), W_h_b.reshape(1, H))


# speedup vs baseline: 1.2209x; 1.2209x over previous
import functools

import jax
import jax.numpy as jnp
from jax import lax
from jax.experimental import pallas as pl
from jax.experimental.pallas import tpu as pltpu
from jax.experimental.pallas import tpu_sc as plsc

E = 160000
MNN = 6
IN = 128
H = 128
HEADS = 8
DPH = 16

NEG = -1e18
SLOPE = 0.01


_B1 = 1600


def _lrelu(v):
    return jnp.where(v >= 0, v, SLOPE * v)


def _tc1_body(x_ref, wzx_ref, wr_ref, whx_ref, whh_ref, wk_ref, wv_ref, wq_ref,
              cz_ref, cr_ref, whb_ref, wkb_ref, wvb_ref, wqb_ref, sqb_ref,
              aq_ref, ak_ref, seg_ref, v0_ref, sks_ref, sq_ref):
    x = x_ref[...]
    f32 = jnp.float32
    dot = functools.partial(jnp.dot, preferred_element_type=f32)
    wvb = wvb_ref[...]
    z0 = jax.nn.sigmoid(dot(x, wzx_ref[...]) + cz_ref[...])
    r0 = jax.nn.sigmoid(dot(x, wr_ref[...]) + cr_ref[...])
    pre0 = jnp.tanh(dot(x, whx_ref[...]) + dot(r0 * wvb, whh_ref[...]) + whb_ref[...])
    h0 = (1.0 - z0) * wvb + z0 * pre0
    rid = lax.broadcasted_iota(jnp.int32, (x.shape[0], 1), 0) + pl.program_id(0) * _B1
    h0 = jnp.where(rid == 0, f32(0.0), h0)
    v0_ref[...] = dot(h0, wv_ref[...]) + wvb
    k0 = dot(h0, wk_ref[...]) + wkb_ref[...]
    sk = dot(_lrelu(k0) * ak_ref[...], seg_ref[...])
    s0 = jnp.sum(h0, axis=1, keepdims=True)
    pad7 = jnp.zeros((x.shape[0], 7), f32)
    sks_ref[...] = jnp.concatenate([sk, s0, pad7], axis=1)
    q = dot(x, wq_ref[...]) + wqb_ref[...]
    sq = dot(_lrelu(q) * aq_ref[...], seg_ref[...]) + sqb_ref[...]
    pad8 = jnp.zeros((x.shape[0], 8), f32)
    sq_ref[...] = jnp.concatenate([sq, pad8], axis=1)


def _tc1(x, wzxT, wrT, whxT, whhT, wkT, wvT, wqT, cz, cr, whb, wkb, wvb, wqb,
         sqb, aq, ak, seg):
    nblk = E // _B1
    row_blk = pl.BlockSpec((_B1, H), lambda i: (i, 0))
    full = lambda shape: pl.BlockSpec(shape, lambda i: tuple(0 for _ in shape))
    return pl.pallas_call(
        _tc1_body,
        grid=(nblk,),
        in_specs=[row_blk] + [full((H, H))] * 7 + [full((1, H))] * 6
                 + [full((1, HEADS))] + [full((1, H))] * 2 + [full((H, HEADS))],
        out_specs=[row_blk,
                   pl.BlockSpec((_B1, 16), lambda i: (i, 0)),
                   pl.BlockSpec((_B1, 16), lambda i: (i, 0))],
        out_shape=[jax.ShapeDtypeStruct((E, H), jnp.float32),
                   jax.ShapeDtypeStruct((E, 16), jnp.float32),
                   jax.ShapeDtypeStruct((E, 16), jnp.float32)],
    )(x, wzxT, wrT, whxT, whhT, wkT, wvT, wqT, cz, cr, whb, wkb, wvb, wqb,
      sqb, aq, ak, seg)


_NC, _NS = 2, 16
_NW = _NC * _NS
_R = 16
_EPC = _R * MNN
_NCHUNK = E // _R


_G = 8
_BR = _G * _R
_BE = _G * _EPC
_NBATCH = E // _BR


def _sc_body(bg_hbm, v0_hbm, sks_hbm, sq_hbm, out_hbm,
             idx_v, sq_v, outb_v, vg_v, sksg_v, sems):
    cid = lax.axis_index("c")
    sid = lax.axis_index("s")
    wid = sid * _NC + cid
    nb = (_NBATCH - wid + _NW - 1) // _NW
    i16 = lax.iota(jnp.int32, 16)
    rows = [i16 * MNN + j for j in range(MNN)]
    col_s0 = jnp.full((16,), HEADS, jnp.int32)

    def issue(g, p):
        ids = idx_v.at[pl.ds(g * _EPC, _EPC)]
        cv = pltpu.async_copy(v0_hbm.at[ids], vg_v.at[p], sems.at[2 * p])
        cs = pltpu.async_copy(sks_hbm.at[ids], sksg_v.at[p], sems.at[2 * p + 1])
        return cv, cs

    def drain(g, p):
        ids = idx_v.at[pl.ds(g * _EPC, _EPC)]
        pltpu.make_async_copy(v0_hbm.at[ids], vg_v.at[p], sems.at[2 * p]).wait()
        pltpu.make_async_copy(sks_hbm.at[ids], sksg_v.at[p], sems.at[2 * p + 1]).wait()

    _dnums = lax.GatherDimensionNumbers(offset_dims=(), collapsed_slice_dims=(0,),
                                        start_index_map=(0,))

    def splat(vec, lane):
        idx = jnp.full((16, 1), lane, jnp.int32)
        return lax.gather(vec, idx, _dnums, (1,),
                          mode=lax.GatherScatterMode.PROMISE_IN_BOUNDS)

    cols = [jnp.int32(h * DPH) + i16 for h in range(HEADS)]

    def compute(g, p):
        vg = vg_v.at[p]
        sksg = sksg_v.at[p]

        def one_row(r):
            row = g * _R + r
            rowspl = jnp.broadcast_to(row, (16,)).astype(jnp.int32)
            sqv = plsc.load_gather(sq_v, [rowspl, i16])
            erows = [jnp.broadcast_to(r * MNN + j, (16,)).astype(jnp.int32)
                     for j in range(MNN)]
            kv = [plsc.load_gather(sksg, [erows[j], i16]) for j in range(MNN)]
            sc = [jnp.where(splat(kv[j], HEADS) == 0.0, NEG, sqv + kv[j])
                  for j in range(MNN)]
            m = sc[0]
            for j in range(1, MNN):
                m = jnp.maximum(m, sc[j])
            es = [jnp.exp(s - m) for s in sc]
            tot = es[0]
            for j in range(1, MNN):
                tot = tot + es[j]
            inv = 1.0 / tot
            wv = [e * inv for e in es]
            for h in range(HEADS):
                acc = None
                for j in range(MNN):
                    wsp = splat(wv[j], h)
                    vvec = plsc.load_gather(vg, [erows[j], cols[h]])
                    acc = wsp * vvec if acc is None else acc + wsp * vvec
                plsc.store_scatter(outb_v, [rowspl, cols[h]], acc)

        def row_body(rr, carry3):
            one_row(rr * 2)
            one_row(rr * 2 + 1)
            return carry3

        lax.fori_loop(0, _R // 2, row_body, 0)

    def batch(b, carry):
        bid = wid + _NW * b
        base_row = bid * _BR
        base_edge = bid * _BE
        pltpu.sync_copy(bg_hbm.at[pl.ds(base_edge, _BE)], idx_v)
        pltpu.sync_copy(sq_hbm.at[pl.ds(base_row, _BR), :], sq_v)
        issue(0, 0)

        def gpair(gp, carry2):
            g0 = gp * 2
            issue(g0 + 1, 1)
            drain(g0, 0)
            compute(g0, 0)

            @pl.when(gp < _G // 2 - 1)
            def _():
                issue(g0 + 2, 0)

            drain(g0 + 1, 1)
            compute(g0 + 1, 1)
            return carry2

        lax.fori_loop(0, _G // 2, gpair, 0)
        pltpu.sync_copy(outb_v, out_hbm.at[pl.ds(base_row, _BR), :])
        return carry

    lax.fori_loop(0, nb, batch, 0)


def _sc_attn(bg_flat, v0, sks, sq):
    mesh = plsc.VectorSubcoreMesh(core_axis_name="c", subcore_axis_name="s",
                                  num_cores=_NC, num_subcores=_NS)
    f = pl.kernel(
        _sc_body,
        out_type=jax.ShapeDtypeStruct((E, H), jnp.float32),
        mesh=mesh,
        compiler_params=pltpu.CompilerParams(needs_layout_passes=False,
                                             use_tc_tiling_on_sc=False),
        scratch_types=[
            pltpu.VMEM((_BE,), jnp.int32),
            pltpu.VMEM((_BR, 16), jnp.float32),
            pltpu.VMEM((_BR, H), jnp.float32),
            pltpu.VMEM((2, _EPC, H), jnp.float32),
            pltpu.VMEM((2, _EPC, 16), jnp.float32),
            pltpu.SemaphoreType.DMA((4,)),
        ],
    )
    return f(bg_flat, v0, sks, sq)


def _tc2_body(x_ref, sh_ref, wzx_ref, wzh_ref, wr_ref, ur_ref, whx_ref,
              whh_ref, wzb_ref, urb_ref, whb_ref, out_ref):
    x = x_ref[...]
    sh = sh_ref[...]
    dot = functools.partial(jnp.dot, preferred_element_type=jnp.float32)
    z = jax.nn.sigmoid(dot(x, wzx_ref[...]) + dot(sh, wzh_ref[...]) + wzb_ref[...])
    r = jax.nn.sigmoid(dot(x, wr_ref[...]) + dot(sh, ur_ref[...]) + urb_ref[...])
    pre = jnp.tanh(dot(x, whx_ref[...]) + dot(r * sh, whh_ref[...]) + whb_ref[...])
    h = (1.0 - z) * sh + z * pre
    rid = lax.broadcasted_iota(jnp.int32, (x.shape[0], 1), 0) + pl.program_id(0) * _B1
    out_ref[...] = jnp.where(rid == 0, jnp.float32(0.0), h)


def _tc2(x, sum_h, wzxT, wzhT, wrT, urT, whxT, whhT, wzb, urb, whb):
    nblk = E // _B1
    row_blk = pl.BlockSpec((_B1, H), lambda i: (i, 0))
    full = lambda shape: pl.BlockSpec(shape, lambda i: tuple(0 for _ in shape))
    return pl.pallas_call(
        _tc2_body,
        grid=(nblk,),
        in_specs=[row_blk, row_blk] + [full((H, H))] * 6 + [full((1, H))] * 3,
        out_specs=row_blk,
        out_shape=jax.ShapeDtypeStruct((E, H), jnp.float32),
    )(x, sum_h, wzxT, wzhT, wrT, urT, whxT, whhT, wzb, urb, whb)


def kernel(fmess, bgraph, W_z_w, W_z_b, W_r_w, U_r_w, U_r_b, W_h_w, W_h_b,
           attn_alpha, attn_bias, Wq_w, Wq_b, Wk_w, Wk_b, Wv_w, Wv_b):
    x = fmess
    f32 = jnp.float32

    Wz_x, Wz_h = W_z_w[:, :IN], W_z_w[:, IN:]
    Wh_x, Wh_h = W_h_w[:, :IN], W_h_w[:, IN:]
    alpha = attn_alpha.reshape(HEADS, 2 * DPH)
    aq = alpha[:, :DPH].reshape(1, H)
    ak = alpha[:, DPH:].reshape(1, H)
    bias = attn_bias.reshape(1, HEADS)

    col = jnp.arange(H)
    seg = (col[:, None] // DPH == jnp.arange(HEADS)[None, :]).astype(f32)

    cz = (Wv_b @ Wz_h.T + W_z_b).reshape(1, H)
    cr = (Wv_b @ U_r_w.T + U_r_b).reshape(1, H)

    v0, sks, sq = _tc1(
        x, Wz_x.T, W_r_w.T, Wh_x.T, Wh_h.T, Wk_w.T, Wv_w.T, Wq_w.T,
        cz, cr, W_h_b.reshape(1, H), Wk_b.reshape(1, H), Wv_b.reshape(1, H),
        Wq_b.reshape(1, H), bias, aq, ak, seg)

    sum_h = _sc_attn(bgraph.reshape(E * MNN), v0, sks, sq)

    return _tc2(x, sum_h, Wz_x.T, Wz_h.T, W_r_w.T, U_r_w.T, Wh_x.T, Wh_h.T,
                W_z_b.reshape(1, H), U_r_b.reshape(1, H), W_h_b.reshape(1, H))


# speedup vs baseline: 1.4352x; 1.1755x over previous
import functools

import jax
import jax.numpy as jnp
from jax import lax
from jax.experimental import pallas as pl
from jax.experimental.pallas import tpu as pltpu
from jax.experimental.pallas import tpu_sc as plsc

E = 160000
MNN = 6
IN = 128
H = 128
HEADS = 8
DPH = 16

NEG = -1e18
SLOPE = 0.01


_B1 = 1600


def _lrelu(v):
    return jnp.where(v >= 0, v, SLOPE * v)


def _tc1_body(x_ref, wzx_ref, wr_ref, whx_ref, whh_ref, wk_ref, wv_ref, wq_ref,
              cz_ref, cr_ref, whb_ref, wkb_ref, wvb_ref, wqb_ref, sqb_ref,
              aq_ref, ak_ref, seg_ref, v0_ref, sks_ref, sq_ref):
    x = x_ref[...]
    f32 = jnp.float32
    dot = functools.partial(jnp.dot, preferred_element_type=f32)
    wvb = wvb_ref[...]
    z0 = jax.nn.sigmoid(dot(x, wzx_ref[...]) + cz_ref[...])
    r0 = jax.nn.sigmoid(dot(x, wr_ref[...]) + cr_ref[...])
    pre0 = jnp.tanh(dot(x, whx_ref[...]) + dot(r0 * wvb, whh_ref[...]) + whb_ref[...])
    h0 = (1.0 - z0) * wvb + z0 * pre0
    rid = lax.broadcasted_iota(jnp.int32, (x.shape[0], 1), 0) + pl.program_id(0) * _B1
    h0 = jnp.where(rid == 0, f32(0.0), h0)
    v0 = dot(h0, wv_ref[...]) + wvb
    u = lax.bitcast_convert_type(v0, jnp.uint32)
    r = (u + jnp.uint32(0x7FFF) + ((u >> 16) & jnp.uint32(1))) >> 16
    v0_ref[...] = lax.bitcast_convert_type(
        r[:, :64] | (r[:, 64:] << 16), jnp.int32)
    k0 = dot(h0, wk_ref[...]) + wkb_ref[...]
    sk = dot(_lrelu(k0) * ak_ref[...], seg_ref[...])
    s0 = jnp.sum(h0, axis=1, keepdims=True)
    sk = jnp.where(s0 == 0.0, f32(-1e30), sk)
    pad8 = jnp.zeros((x.shape[0], 8), f32)
    sks_ref[...] = jnp.concatenate([sk, pad8], axis=1)
    q = dot(x, wq_ref[...]) + wqb_ref[...]
    sq = dot(_lrelu(q) * aq_ref[...], seg_ref[...]) + sqb_ref[...]
    pad8 = jnp.zeros((x.shape[0], 8), f32)
    sq_ref[...] = jnp.concatenate([sq, pad8], axis=1)


def _tc1(x, wzxT, wrT, whxT, whhT, wkT, wvT, wqT, cz, cr, whb, wkb, wvb, wqb,
         sqb, aq, ak, seg):
    nblk = E // _B1
    row_blk = pl.BlockSpec((_B1, H), lambda i: (i, 0))
    full = lambda shape: pl.BlockSpec(shape, lambda i: tuple(0 for _ in shape))
    return pl.pallas_call(
        _tc1_body,
        grid=(nblk,),
        in_specs=[row_blk] + [full((H, H))] * 7 + [full((1, H))] * 6
                 + [full((1, HEADS))] + [full((1, H))] * 2 + [full((H, HEADS))],
        out_specs=[pl.BlockSpec((_B1, 64), lambda i: (i, 0)),
                   pl.BlockSpec((_B1, 16), lambda i: (i, 0)),
                   pl.BlockSpec((_B1, 16), lambda i: (i, 0))],
        out_shape=[jax.ShapeDtypeStruct((E, 64), jnp.int32),
                   jax.ShapeDtypeStruct((E, 16), jnp.float32),
                   jax.ShapeDtypeStruct((E, 16), jnp.float32)],
    )(x, wzxT, wrT, whxT, whhT, wkT, wvT, wqT, cz, cr, whb, wkb, wvb, wqb,
      sqb, aq, ak, seg)


_NC, _NS = 2, 16
_NW = _NC * _NS
_R = 16
_EPC = _R * MNN
_NCHUNK = E // _R


_G = 16
_BR = _G * _R
_BE = _G * _EPC
_NBATCH = E // _BR


def _sc_body(bg_hbm, v0_hbm, sks_hbm, sq_hbm, out_hbm,
             idx_v, sq_v, outb_v, vg_v, sksg_v, sems):
    cid = lax.axis_index("c")
    sid = lax.axis_index("s")
    wid = sid * _NC + cid
    nb = (_NBATCH - wid + _NW - 1) // _NW
    i16 = lax.iota(jnp.int32, 16)
    rows = [i16 * MNN + j for j in range(MNN)]
    col_s0 = jnp.full((16,), HEADS, jnp.int32)

    def issue(g, p):
        ids = idx_v.at[pl.ds(g * _EPC, _EPC)]
        cv = pltpu.async_copy(v0_hbm.at[ids], vg_v.at[p], sems.at[2 * p])
        cs = pltpu.async_copy(sks_hbm.at[ids], sksg_v.at[p], sems.at[2 * p + 1])
        return cv, cs

    def drain(g, p):
        ids = idx_v.at[pl.ds(g * _EPC, _EPC)]
        pltpu.make_async_copy(v0_hbm.at[ids], vg_v.at[p], sems.at[2 * p]).wait()
        pltpu.make_async_copy(sks_hbm.at[ids], sksg_v.at[p], sems.at[2 * p + 1]).wait()

    _dnums = lax.GatherDimensionNumbers(offset_dims=(), collapsed_slice_dims=(0,),
                                        start_index_map=(0,))

    def splat(vec, lane):
        idx = jnp.full((16, 1), lane, jnp.int32)
        return lax.gather(vec, idx, _dnums, (1,),
                          mode=lax.GatherScatterMode.PROMISE_IN_BOUNDS)

    cols = [jnp.int32(h * DPH) + i16 for h in range(HEADS)]

    def compute(g, p):
        vg = vg_v.at[p]
        sksg = sksg_v.at[p]

        def one_row(r):
            row = g * _R + r
            rowspl = jnp.broadcast_to(row, (16,)).astype(jnp.int32)
            sqv = plsc.load_gather(sq_v, [rowspl, i16])
            erows = [jnp.broadcast_to(r * MNN + j, (16,)).astype(jnp.int32)
                     for j in range(MNN)]
            sc = [sqv + plsc.load_gather(sksg, [erows[j], i16])
                  for j in range(MNN)]
            m = sc[0]
            for j in range(1, MNN):
                m = jnp.maximum(m, sc[j])
            es = [jnp.exp(s - m) for s in sc]
            tot = es[0]
            for j in range(1, MNN):
                tot = tot + es[j]
            inv = 1.0 / tot
            wv = [e * inv for e in es]
            for q in range(4):
                acc_lo = acc_hi = None
                for j in range(MNN):
                    w = plsc.load_gather(vg, [erows[j], cols[q]])
                    lo = plsc.bitcast(w << 16, jnp.float32)
                    hi = plsc.bitcast(w & -65536, jnp.float32)
                    wlo = splat(wv[j], q)
                    whi = splat(wv[j], q + 4)
                    if acc_lo is None:
                        acc_lo = wlo * lo
                        acc_hi = whi * hi
                    else:
                        acc_lo = acc_lo + wlo * lo
                        acc_hi = acc_hi + whi * hi
                plsc.store_scatter(outb_v, [rowspl, cols[q]], acc_lo)
                plsc.store_scatter(outb_v, [rowspl, cols[q + 4]], acc_hi)

        def row_body(rr, carry3):
            one_row(rr * 2)
            one_row(rr * 2 + 1)
            return carry3

        lax.fori_loop(0, _R // 2, row_body, 0)

    def batch(b, carry):
        bid = wid + _NW * b
        base_row = bid * _BR
        base_edge = bid * _BE
        pltpu.sync_copy(bg_hbm.at[pl.ds(base_edge, _BE)], idx_v)
        pltpu.sync_copy(sq_hbm.at[pl.ds(base_row, _BR), :], sq_v)
        issue(0, 0)

        def gpair(gp, carry2):
            g0 = gp * 2
            issue(g0 + 1, 1)
            drain(g0, 0)
            compute(g0, 0)

            @pl.when(gp < _G // 2 - 1)
            def _():
                issue(g0 + 2, 0)

            drain(g0 + 1, 1)
            compute(g0 + 1, 1)
            return carry2

        lax.fori_loop(0, _G // 2, gpair, 0)
        pltpu.sync_copy(outb_v, out_hbm.at[pl.ds(base_row, _BR), :])
        return carry

    lax.fori_loop(0, nb, batch, 0)


def _sc_attn(bg_flat, v0, sks, sq):
    mesh = plsc.VectorSubcoreMesh(core_axis_name="c", subcore_axis_name="s",
                                  num_cores=_NC, num_subcores=_NS)
    f = pl.kernel(
        _sc_body,
        out_type=jax.ShapeDtypeStruct((E, H), jnp.float32),
        mesh=mesh,
        compiler_params=pltpu.CompilerParams(needs_layout_passes=False,
                                             use_tc_tiling_on_sc=False),
        scratch_types=[
            pltpu.VMEM((_BE,), jnp.int32),
            pltpu.VMEM((_BR, 16), jnp.float32),
            pltpu.VMEM((_BR, H), jnp.float32),
            pltpu.VMEM((2, _EPC, 64), jnp.int32),
            pltpu.VMEM((2, _EPC, 16), jnp.float32),
            pltpu.SemaphoreType.DMA((4,)),
        ],
    )
    return f(bg_flat, v0, sks, sq)


def _tc2_body(x_ref, sh_ref, wzx_ref, wzh_ref, wr_ref, ur_ref, whx_ref,
              whh_ref, wzb_ref, urb_ref, whb_ref, out_ref):
    x = x_ref[...]
    sh = sh_ref[...]
    dot = functools.partial(jnp.dot, preferred_element_type=jnp.float32)
    z = jax.nn.sigmoid(dot(x, wzx_ref[...]) + dot(sh, wzh_ref[...]) + wzb_ref[...])
    r = jax.nn.sigmoid(dot(x, wr_ref[...]) + dot(sh, ur_ref[...]) + urb_ref[...])
    pre = jnp.tanh(dot(x, whx_ref[...]) + dot(r * sh, whh_ref[...]) + whb_ref[...])
    h = (1.0 - z) * sh + z * pre
    rid = lax.broadcasted_iota(jnp.int32, (x.shape[0], 1), 0) + pl.program_id(0) * _B1
    out_ref[...] = jnp.where(rid == 0, jnp.float32(0.0), h)


def _tc2(x, sum_h, wzxT, wzhT, wrT, urT, whxT, whhT, wzb, urb, whb):
    nblk = E // _B1
    row_blk = pl.BlockSpec((_B1, H), lambda i: (i, 0))
    full = lambda shape: pl.BlockSpec(shape, lambda i: tuple(0 for _ in shape))
    return pl.pallas_call(
        _tc2_body,
        grid=(nblk,),
        in_specs=[row_blk, row_blk] + [full((H, H))] * 6 + [full((1, H))] * 3,
        out_specs=row_blk,
        out_shape=jax.ShapeDtypeStruct((E, H), jnp.float32),
    )(x, sum_h, wzxT, wzhT, wrT, urT, whxT, whhT, wzb, urb, whb)


def kernel(fmess, bgraph, W_z_w, W_z_b, W_r_w, U_r_w, U_r_b, W_h_w, W_h_b,
           attn_alpha, attn_bias, Wq_w, Wq_b, Wk_w, Wk_b, Wv_w, Wv_b):
    x = fmess
    f32 = jnp.float32

    Wz_x, Wz_h = W_z_w[:, :IN], W_z_w[:, IN:]
    Wh_x, Wh_h = W_h_w[:, :IN], W_h_w[:, IN:]
    alpha = attn_alpha.reshape(HEADS, 2 * DPH)
    aq = alpha[:, :DPH].reshape(1, H)
    ak = alpha[:, DPH:].reshape(1, H)
    bias = attn_bias.reshape(1, HEADS)

    col = jnp.arange(H)
    seg = (col[:, None] // DPH == jnp.arange(HEADS)[None, :]).astype(f32)

    cz = (Wv_b @ Wz_h.T + W_z_b).reshape(1, H)
    cr = (Wv_b @ U_r_w.T + U_r_b).reshape(1, H)

    v0, sks, sq = _tc1(
        x, Wz_x.T, W_r_w.T, Wh_x.T, Wh_h.T, Wk_w.T, Wv_w.T, Wq_w.T,
        cz, cr, W_h_b.reshape(1, H), Wk_b.reshape(1, H), Wv_b.reshape(1, H),
        Wq_b.reshape(1, H), bias, aq, ak, seg)

    sum_h = _sc_attn(bgraph.reshape(E * MNN), v0, sks, sq)

    return _tc2(x, sum_h, Wz_x.T, Wz_h.T, W_r_w.T, U_r_w.T, Wh_x.T, Wh_h.T,
                W_z_b.reshape(1, H), U_r_b.reshape(1, H), W_h_b.reshape(1, H))
